# Initial kernel scaffold; baseline (speedup 1.0000x reference)
import jax
import jax.numpy as jnp
from jax.experimental import pallas as pl


def kernel(inputs, vals):
    raise NotImplementedError("write your pallas kernel here")


# speedup vs baseline: 1.7184x; 1.7184x over previous
import functools

import jax
import jax.numpy as jnp
from jax import lax
from jax.experimental import pallas as pl
from jax.experimental.pallas import tpu as pltpu
from jax.experimental.pallas import tpu_sc as plsc

N = 1000
NBP = 1024
TBL = N * NBP
NUM_SAMPLES = 2_000_000
NC, NS, L = 2, 16, 16
NW = NC * NS
S_W = NUM_SAMPLES // NW
CH = 4096
N_FULL = S_W // CH
LC = S_W - N_FULL * CH
LC_FULL_G = LC // L
LC_REM = LC - LC_FULL_G * L
G_FULL = CH // L
CBL = (LC_FULL_G + 1) * L
ZB = 8000
Z_SPAN = TBL // NS
DUMMY = NBP - 1


def _hist_body(inp, out, table, inbuf, cbuf, ones1d, cbuf_l, ones_l, zbuf):
    cid = lax.axis_index("c")
    sid = lax.axis_index("s")
    wid = cid * NS + sid
    word_base = wid * (2 * S_W)
    iota = lax.iota(jnp.int32, L)
    iota2 = iota * 2
    zeros16 = jnp.zeros((L,), jnp.float32)
    ones16 = jnp.ones((L,), jnp.float32)
    dummy16 = jnp.full((L,), DUMMY, jnp.int32)

    def fill_z(i, _):
        zbuf[pl.ds(i * L, L)] = zeros16
        return _

    lax.fori_loop(0, ZB // L, fill_z, None)

    def fill_ones(i, _):
        ones1d[pl.ds(i * L, L)] = ones16
        return _

    lax.fori_loop(0, CH // L, fill_ones, None)

    def fill_tail(i, _):
        ones_l[pl.ds(i * L, L)] = ones16
        cbuf_l[pl.ds(i * L, L)] = dummy16
        return _

    lax.fori_loop(0, CBL // L, fill_tail, None)

    for j in range(Z_SPAN // ZB):
        pltpu.sync_copy(zbuf, table.at[pl.ds(sid * Z_SPAN + j * ZB, ZB)])
    plsc.subcore_barrier()

    def group(g, _):
        ia = iota2 + g * (2 * L)
        a = plsc.load_gather(inbuf, [ia])
        b = plsc.load_gather(inbuf, [ia + 1])
        cbuf[pl.ds(g * L, L)] = (a << 10) | b
        return _

    def chunk(ch, _):
        off = word_base + ch * (2 * CH)
        pltpu.sync_copy(inp.at[pl.ds(off, 2 * CH)], inbuf)
        lax.fori_loop(0, G_FULL, group, None)
        pltpu.sync_copy(ones1d, table.at[cbuf], add=True)
        return _

    lax.fori_loop(0, N_FULL, chunk, None)

    off = word_base + N_FULL * (2 * CH)
    pltpu.sync_copy(inp.at[pl.ds(off, 2 * LC)], inbuf.at[pl.ds(0, 2 * LC)])

    def group_l(g, _):
        ia = iota2 + g * (2 * L)
        a = plsc.load_gather(inbuf, [ia])
        b = plsc.load_gather(inbuf, [ia + 1])
        cbuf_l[pl.ds(g * L, L)] = (a << 10) | b
        return _

    lax.fori_loop(0, LC_FULL_G, group_l, None)
    ia = iota2 + LC_FULL_G * (2 * L)
    a = plsc.load_gather(inbuf, [ia])
    b = plsc.load_gather(inbuf, [ia + 1])
    c = jnp.where(iota < LC_REM, (a << 10) | b, dummy16)
    cbuf_l[pl.ds(LC_FULL_G * L, L)] = c
    pltpu.sync_copy(ones_l, table.at[cbuf_l], add=True)

    plsc.subcore_barrier()
    pltpu.sync_copy(
        table.at[pl.ds(sid * Z_SPAN, Z_SPAN)],
        out.at[pl.ds(cid * TBL + sid * Z_SPAN, Z_SPAN)],
    )


_sc_hist = functools.partial(
    pl.kernel,
    out_type=jax.ShapeDtypeStruct((NC * TBL,), jnp.float32),
    mesh=plsc.VectorSubcoreMesh(core_axis_name="c", subcore_axis_name="s"),
    compiler_params=pltpu.CompilerParams(needs_layout_passes=False),
    scratch_types=[
        pltpu.VMEM_SHARED((TBL,), jnp.float32),
        pltpu.VMEM((2 * CH,), jnp.int32),
        pltpu.VMEM((CH,), jnp.int32),
        pltpu.VMEM((CH,), jnp.float32),
        pltpu.VMEM((CBL,), jnp.int32),
        pltpu.VMEM((CBL,), jnp.float32),
        pltpu.VMEM((ZB,), jnp.float32),
    ],
)(_hist_body)


def _finalize_body(p_ref, o_ref, a_ref):
    s = p_ref[0] + p_ref[1]
    v = s[:, :N]
    rs = jnp.sum(v, axis=1, keepdims=True)
    a_ref[...] = rs * (1.0 / NUM_SAMPLES)
    o_ref[...] = v / jnp.maximum(rs, 1.0)


def _finalize(p3):
    return pl.pallas_call(
        _finalize_body,
        grid=(N // 8,),
        in_specs=[pl.BlockSpec((NC, 8, NBP), lambda i: (0, i, 0))],
        out_specs=[
            pl.BlockSpec((8, N), lambda i: (i, 0)),
            pl.BlockSpec((8, 1), lambda i: (i, 0)),
        ],
        out_shape=[
            jax.ShapeDtypeStruct((N, N), jnp.float32),
            jax.ShapeDtypeStruct((N, 1), jnp.float32),
        ],
    )(p3)


def kernel(inputs, vals):
    del vals
    flat = inputs.reshape(-1)
    part = _sc_hist(flat)
    pi_b_a, pi_a = _finalize(part.reshape(NC, N, NBP))
    return pi_a.reshape(N), pi_b_a


# speedup vs baseline: 26.0968x; 15.1864x over previous
import functools

import jax
import jax.numpy as jnp
from jax import lax
from jax.experimental import pallas as pl
from jax.experimental.pallas import tpu as pltpu
from jax.experimental.pallas import tpu_sc as plsc

N = 1000
NBP = 1024
TBL = N * NBP
NUM_SAMPLES = 2_000_000
NC, NS, L = 2, 16, 16
NW = NC * NS
NB = NUM_SAMPLES // 128
BPW = NB // NW
REM = NB - BPW * NW
CHB = 32
N_FULL = BPW // CHB
TAILB = BPW - N_FULL * CHB + REM
CH = CHB * 128
CBL = TAILB * 128
ZB = 8000
Z_SPAN = TBL // NS
DUMMY = NBP - 1


def _hist_body(inp, out, table, inbuf, cbuf, ones1d, cbuf_l, ones_l, zbuf):
    cid = lax.axis_index("c")
    sid = lax.axis_index("s")
    wid = cid * NS + sid
    base = wid * BPW
    iota = lax.iota(jnp.int32, L)
    zeros16 = jnp.zeros((L,), jnp.float32)
    ones16 = jnp.ones((L,), jnp.float32)
    dummy16 = jnp.full((L,), DUMMY, jnp.int32)

    def fill_z(i, _):
        zbuf[pl.ds(i * L, L)] = zeros16
        return _

    lax.fori_loop(0, ZB // L, fill_z, None)

    def fill_ones(i, _):
        ones1d[pl.ds(i * L, L)] = ones16
        return _

    lax.fori_loop(0, CH // L, fill_ones, None)

    def fill_ones_l(i, _):
        ones_l[pl.ds(i * L, L)] = ones16
        return _

    lax.fori_loop(0, CBL // L, fill_ones_l, None)

    for j in range(Z_SPAN // ZB):
        pltpu.sync_copy(zbuf, table.at[pl.ds(sid * Z_SPAN + j * ZB, ZB)])
    plsc.subcore_barrier()

    def blocks(buf, blk, cdst, cbase):
        for gg in range(8):
            va = buf[blk, 0, pl.ds(gg * L, L)]
            vb = buf[blk, 1, pl.ds(gg * L, L)]
            cdst[pl.ds(cbase + gg * L, L)] = (va << 10) | vb

    def chunk(ch, _):
        pltpu.sync_copy(inp.at[pl.ds(base + ch * CHB, CHB)], inbuf)

        def blk_body(blk, _):
            blocks(inbuf, blk, cbuf, blk * 128)
            return _

        lax.fori_loop(0, CHB, blk_body, None)
        pltpu.sync_copy(ones1d, table.at[cbuf], add=True)
        return _

    lax.fori_loop(0, N_FULL, chunk, None)

    nreal = jnp.where(wid == NW - 1, TAILB, TAILB - REM)
    pltpu.sync_copy(inp.at[pl.ds(base + N_FULL * CHB, TAILB)], inbuf.at[pl.ds(0, TAILB)])

    def blk_body_l(blk, _):
        for gg in range(8):
            va = inbuf[blk, 0, pl.ds(gg * L, L)]
            vb = inbuf[blk, 1, pl.ds(gg * L, L)]
            c = jnp.where(blk < nreal, (va << 10) | vb, dummy16)
            cbuf_l[pl.ds(blk * 128 + gg * L, L)] = c
        return _

    lax.fori_loop(0, TAILB, blk_body_l, None)
    pltpu.sync_copy(ones_l, table.at[cbuf_l], add=True)

    plsc.subcore_barrier()
    pltpu.sync_copy(
        table.at[pl.ds(sid * Z_SPAN, Z_SPAN)],
        out.at[pl.ds(cid * TBL + sid * Z_SPAN, Z_SPAN)],
    )


_sc_hist = functools.partial(
    pl.kernel,
    out_type=jax.ShapeDtypeStruct((NC * TBL,), jnp.float32),
    mesh=plsc.VectorSubcoreMesh(core_axis_name="c", subcore_axis_name="s"),
    compiler_params=pltpu.CompilerParams(needs_layout_passes=False),
    scratch_types=[
        pltpu.VMEM_SHARED((TBL,), jnp.float32),
        pltpu.VMEM((CHB, 2, 128), jnp.int32),
        pltpu.VMEM((CH,), jnp.int32),
        pltpu.VMEM((CH,), jnp.float32),
        pltpu.VMEM((CBL,), jnp.int32),
        pltpu.VMEM((CBL,), jnp.float32),
        pltpu.VMEM((ZB,), jnp.float32),
    ],
)(_hist_body)


def _finalize_body(p_ref, o_ref, a_ref):
    s = p_ref[0] + p_ref[1]
    v = s[:, :N]
    rs = jnp.sum(v, axis=1, keepdims=True)
    a_ref[...] = rs * (1.0 / NUM_SAMPLES)
    o_ref[...] = v / jnp.maximum(rs, 1.0)


def _finalize(p3):
    return pl.pallas_call(
        _finalize_body,
        grid=(N // 8,),
        in_specs=[pl.BlockSpec((NC, 8, NBP), lambda i: (0, i, 0))],
        out_specs=[
            pl.BlockSpec((8, N), lambda i: (i, 0)),
            pl.BlockSpec((8, 1), lambda i: (i, 0)),
        ],
        out_shape=[
            jax.ShapeDtypeStruct((N, N), jnp.float32),
            jax.ShapeDtypeStruct((N, 1), jnp.float32),
        ],
    )(p3)


def kernel(inputs, vals):
    del vals
    view = inputs.reshape(NB, 128, 2).transpose(0, 2, 1)
    part = _sc_hist(view)
    pi_b_a, pi_a = _finalize(part.reshape(NC, N, NBP))
    return pi_a.reshape(N), pi_b_a


# speedup vs baseline: 38.8495x; 1.4887x over previous
import functools

import jax
import jax.numpy as jnp
from jax import lax
from jax.experimental import pallas as pl
from jax.experimental.pallas import tpu as pltpu
from jax.experimental.pallas import tpu_sc as plsc

N = 1000
NBP = 1024
TBL = N * NBP
NUM_SAMPLES = 2_000_000
NC, NS, L = 2, 16, 16
NW = NC * NS
NB = NUM_SAMPLES // 128
BPW = NB // NW
REM = NB - BPW * NW
CHB = 32
N_FULL = BPW // CHB
TAILB = BPW - N_FULL * CHB + REM
CH = CHB * 128
CBL = TAILB * 128
ZB = 8000
Z_SPAN = TBL // NS
DUMMY = NBP - 1


def _hist_body(inp, out, table, inbuf, cbuf, ones1d, cbuf_l, ones_l, zbuf):
    cid = lax.axis_index("c")
    sid = lax.axis_index("s")
    wid = cid * NS + sid
    base = wid * BPW
    iota = lax.iota(jnp.int32, L)
    zeros16 = jnp.zeros((L,), jnp.float32)
    ones16 = jnp.ones((L,), jnp.float32)
    dummy16 = jnp.full((L,), DUMMY, jnp.int32)

    def fill_z(i, _):
        zbuf[pl.ds(i * L, L)] = zeros16
        return _

    lax.fori_loop(0, ZB // L, fill_z, None)

    def fill_ones(i, _):
        ones1d[pl.ds(i * L, L)] = ones16
        return _

    lax.fori_loop(0, CH // L, fill_ones, None)

    def fill_ones_l(i, _):
        ones_l[pl.ds(i * L, L)] = ones16
        return _

    lax.fori_loop(0, CBL // L, fill_ones_l, None)

    for j in range(Z_SPAN // ZB):
        pltpu.sync_copy(zbuf, table.at[pl.ds(sid * Z_SPAN + j * ZB, ZB)])
    plsc.subcore_barrier()

    def blocks(buf, blk, cdst, cbase):
        for gg in range(8):
            va = buf[blk, 0, pl.ds(gg * L, L)]
            vb = buf[blk, 1, pl.ds(gg * L, L)]
            cdst[pl.ds(cbase + gg * L, L)] = (va << 10) | vb

    def chunk(ch, _):
        pltpu.sync_copy(inp.at[pl.ds(base + ch * CHB, CHB)], inbuf)

        def blk_body(blk, _):
            blocks(inbuf, blk, cbuf, blk * 128)
            return _

        lax.fori_loop(0, CHB, blk_body, None)
        pltpu.sync_copy(ones1d, table.at[cbuf], add=True)
        return _

    lax.fori_loop(0, N_FULL, chunk, None)

    nreal = jnp.where(wid == NW - 1, TAILB, TAILB - REM)
    pltpu.sync_copy(inp.at[pl.ds(base + N_FULL * CHB, TAILB)], inbuf.at[pl.ds(0, TAILB)])

    def blk_body_l(blk, _):
        for gg in range(8):
            va = inbuf[blk, 0, pl.ds(gg * L, L)]
            vb = inbuf[blk, 1, pl.ds(gg * L, L)]
            c = jnp.where(blk < nreal, (va << 10) | vb, dummy16)
            cbuf_l[pl.ds(blk * 128 + gg * L, L)] = c
        return _

    lax.fori_loop(0, TAILB, blk_body_l, None)
    pltpu.sync_copy(ones_l, table.at[cbuf_l], add=True)

    plsc.subcore_barrier()
    pltpu.sync_copy(
        table.at[pl.ds(sid * Z_SPAN, Z_SPAN)],
        out.at[pl.ds(cid * TBL + sid * Z_SPAN, Z_SPAN)],
    )


_sc_hist = functools.partial(
    pl.kernel,
    out_type=jax.ShapeDtypeStruct((NC * TBL,), jnp.float32),
    mesh=plsc.VectorSubcoreMesh(core_axis_name="c", subcore_axis_name="s"),
    compiler_params=pltpu.CompilerParams(needs_layout_passes=False),
    scratch_types=[
        pltpu.VMEM_SHARED((TBL,), jnp.float32),
        pltpu.VMEM((CHB, 2, 128), jnp.int32),
        pltpu.VMEM((CH,), jnp.int32),
        pltpu.VMEM((CH,), jnp.float32),
        pltpu.VMEM((CBL,), jnp.int32),
        pltpu.VMEM((CBL,), jnp.float32),
        pltpu.VMEM((ZB,), jnp.float32),
    ],
)(_hist_body)


def _finalize_body(p_ref, o_ref, a_ref):
    s = p_ref[0] + p_ref[1]
    v = s[:, :N]
    rs = jnp.sum(v, axis=1, keepdims=True)
    a_ref[...] = rs * (1.0 / NUM_SAMPLES)
    o_ref[...] = v / jnp.maximum(rs, 1.0)


_FROWS = 200


def _finalize(p3):
    return pl.pallas_call(
        _finalize_body,
        grid=(N // _FROWS,),
        in_specs=[pl.BlockSpec((NC, _FROWS, NBP), lambda i: (0, i, 0))],
        out_specs=[
            pl.BlockSpec((_FROWS, N), lambda i: (i, 0)),
            pl.BlockSpec((_FROWS, 1), lambda i: (i, 0)),
        ],
        out_shape=[
            jax.ShapeDtypeStruct((N, N), jnp.float32),
            jax.ShapeDtypeStruct((N, 1), jnp.float32),
        ],
    )(p3)


def kernel(inputs, vals):
    del vals
    view = inputs.reshape(NB, 128, 2).transpose(0, 2, 1)
    part = _sc_hist(view)
    pi_b_a, pi_a = _finalize(part.reshape(NC, N, NBP))
    return pi_a.reshape(N), pi_b_a


# speedup vs baseline: 51.4924x; 1.3254x over previous
import functools

import jax
import jax.numpy as jnp
from jax import lax
from jax.experimental import pallas as pl
from jax.experimental.pallas import tpu as pltpu
from jax.experimental.pallas import tpu_sc as plsc

N = 1000
NBP = 1024
TBL = N * NBP
NUM_SAMPLES = 2_000_000
NC, NS, L = 2, 16, 16
NW = NC * NS
NB = NUM_SAMPLES // 128
BPW = NB // NW
REM = NB - BPW * NW
CHB = 32
N_FULL = BPW // CHB
TAILB = BPW - N_FULL * CHB + REM
CH = CHB * 128
CBL = TAILB * 128
ZB = 8000
Z_SPAN = TBL // NS
DUMMY = NBP - 1


def _hist_body(inp, out, table, inbuf0, inbuf1, cbuf0, cbuf1, ones1d,
               cbuf_l, ones_l, zbuf, sem_in0, sem_in1, sem_s0, sem_s1,
               sem_l, sem_z):
    cid = lax.axis_index("c")
    sid = lax.axis_index("s")
    wid = cid * NS + sid
    base = wid * BPW
    iota = lax.iota(jnp.int32, L)
    zeros16 = jnp.zeros((L,), jnp.float32)
    ones16 = jnp.ones((L,), jnp.float32)
    dummy16 = jnp.full((L,), DUMMY, jnp.int32)
    inbufs = [inbuf0, inbuf1]
    cbufs = [cbuf0, cbuf1]
    sem_in = [sem_in0, sem_in1]
    sem_s = [sem_s0, sem_s1]

    def fill_z(i, _):
        zbuf[pl.ds(i * L, L)] = zeros16
        return _

    lax.fori_loop(0, ZB // L, fill_z, None)

    def fill_ones(i, _):
        ones1d[pl.ds(i * L, L)] = ones16
        return _

    lax.fori_loop(0, CH // L, fill_ones, None)

    def fill_ones_l(i, _):
        ones_l[pl.ds(i * L, L)] = ones16
        return _

    lax.fori_loop(0, CBL // L, fill_ones_l, None)

    zcopies = [
        pltpu.async_copy(zbuf, table.at[pl.ds(sid * Z_SPAN + j * ZB, ZB)], sem_z)
        for j in range(Z_SPAN // ZB)
    ]
    pending_in = pltpu.async_copy(inp.at[pl.ds(base, CHB)], inbufs[0], sem_in[0])
    for c in zcopies:
        c.wait()
    plsc.subcore_barrier()

    def compute_blocks(src, cdst):
        def blk_body(blk, _):
            for gg in range(8):
                va = src[blk, 0, pl.ds(gg * L, L)]
                vb = src[blk, 1, pl.ds(gg * L, L)]
                cdst[pl.ds(blk * 128 + gg * L, L)] = (va << 10) | vb
            return _

        lax.fori_loop(0, CHB, blk_body, None)

    nreal = jnp.where(wid == NW - 1, TAILB, TAILB - REM)

    def compute_tail(src):
        def blk_body_l(blk, _):
            for gg in range(8):
                va = src[blk, 0, pl.ds(gg * L, L)]
                vb = src[blk, 1, pl.ds(gg * L, L)]
                c = jnp.where(blk < nreal, (va << 10) | vb, dummy16)
                cbuf_l[pl.ds(blk * 128 + gg * L, L)] = c
            return _

        lax.fori_loop(0, TAILB, blk_body_l, None)

    scat = [None, None]
    scat_l = None
    for ch in range(N_FULL + 1):
        cur = ch & 1
        nxt = 1 - cur
        next_in = None
        if ch + 1 < N_FULL:
            next_in = pltpu.async_copy(
                inp.at[pl.ds(base + (ch + 1) * CHB, CHB)], inbufs[nxt], sem_in[nxt]
            )
        elif ch + 1 == N_FULL:
            next_in = pltpu.async_copy(
                inp.at[pl.ds(base + N_FULL * CHB, TAILB)],
                inbufs[nxt].at[pl.ds(0, TAILB)],
                sem_in[nxt],
            )
        pending_in.wait()
        if ch < N_FULL:
            if scat[cur] is not None:
                scat[cur].wait()
            compute_blocks(inbufs[cur], cbufs[cur])
            scat[cur] = pltpu.async_copy(
                ones1d, table.at[cbufs[cur]], sem_s[cur], add=True
            )
        else:
            compute_tail(inbufs[cur])
            scat_l = pltpu.async_copy(ones_l, table.at[cbuf_l], sem_l, add=True)
        pending_in = next_in

    scat[0].wait()
    scat[1].wait()
    scat_l.wait()

    plsc.subcore_barrier()
    pltpu.sync_copy(
        table.at[pl.ds(sid * Z_SPAN, Z_SPAN)],
        out.at[pl.ds(cid * TBL + sid * Z_SPAN, Z_SPAN)],
    )


_sc_hist = functools.partial(
    pl.kernel,
    out_type=jax.ShapeDtypeStruct((NC * TBL,), jnp.float32),
    mesh=plsc.VectorSubcoreMesh(core_axis_name="c", subcore_axis_name="s"),
    compiler_params=pltpu.CompilerParams(needs_layout_passes=False),
    scratch_types=[
        pltpu.VMEM_SHARED((TBL,), jnp.float32),
        pltpu.VMEM((CHB, 2, 128), jnp.int32),
        pltpu.VMEM((CHB, 2, 128), jnp.int32),
        pltpu.VMEM((CH,), jnp.int32),
        pltpu.VMEM((CH,), jnp.int32),
        pltpu.VMEM((CH,), jnp.float32),
        pltpu.VMEM((CBL,), jnp.int32),
        pltpu.VMEM((CBL,), jnp.float32),
        pltpu.VMEM((ZB,), jnp.float32),
        pltpu.SemaphoreType.DMA,
        pltpu.SemaphoreType.DMA,
        pltpu.SemaphoreType.DMA,
        pltpu.SemaphoreType.DMA,
        pltpu.SemaphoreType.DMA,
        pltpu.SemaphoreType.DMA,
    ],
)(_hist_body)


def _finalize_body(p_ref, o_ref, a_ref):
    s = p_ref[0] + p_ref[1]
    v = s[:, :N]
    rs = jnp.sum(v, axis=1, keepdims=True)
    a_ref[...] = rs * (1.0 / NUM_SAMPLES)
    o_ref[...] = v / jnp.maximum(rs, 1.0)


_FROWS = 200


def _finalize(p3):
    return pl.pallas_call(
        _finalize_body,
        grid=(N // _FROWS,),
        in_specs=[pl.BlockSpec((NC, _FROWS, NBP), lambda i: (0, i, 0))],
        out_specs=[
            pl.BlockSpec((_FROWS, N), lambda i: (i, 0)),
            pl.BlockSpec((_FROWS, 1), lambda i: (i, 0)),
        ],
        out_shape=[
            jax.ShapeDtypeStruct((N, N), jnp.float32),
            jax.ShapeDtypeStruct((N, 1), jnp.float32),
        ],
    )(p3)


def kernel(inputs, vals):
    del vals
    view = inputs.reshape(NB, 128, 2).transpose(0, 2, 1)
    part = _sc_hist(view)
    pi_b_a, pi_a = _finalize(part.reshape(NC, N, NBP))
    return pi_a.reshape(N), pi_b_a


# speedup vs baseline: 52.3988x; 1.0176x over previous
import functools

import jax
import jax.numpy as jnp
from jax import lax
from jax.experimental import pallas as pl
from jax.experimental.pallas import tpu as pltpu
from jax.experimental.pallas import tpu_sc as plsc

N = 1000
NBP = 1024
TBL = N * NBP
NUM_SAMPLES = 2_000_000
NC, NS, L = 2, 16, 16
NW = NC * NS
NB = NUM_SAMPLES // 128
BPW = NB // NW
REM = NB - BPW * NW
CHB = 32
N_FULL = BPW // CHB
TAILB = BPW - N_FULL * CHB + REM
CH = CHB * 128
CBL = TAILB * 128
ZB = 8000
Z_SPAN = TBL // NS
DUMMY = (7 << 10) | 127


def _hist_body(inp, out, table, inbuf0, inbuf1, cbuf0, cbuf1, ones1d,
               cbuf_l, ones_l, zbuf, sem_in0, sem_in1, sem_s0, sem_s1,
               sem_l, sem_z):
    cid = lax.axis_index("c")
    sid = lax.axis_index("s")
    wid = cid * NS + sid
    base = wid * BPW
    iota = lax.iota(jnp.int32, L)
    zeros16 = jnp.zeros((L,), jnp.float32)
    ones16 = jnp.ones((L,), jnp.float32)
    dummy16 = jnp.full((L,), DUMMY, jnp.int32)
    inbufs = [inbuf0, inbuf1]
    cbufs = [cbuf0, cbuf1]
    sem_in = [sem_in0, sem_in1]
    sem_s = [sem_s0, sem_s1]

    def fill_z(i, _):
        zbuf[pl.ds(i * L, L)] = zeros16
        return _

    lax.fori_loop(0, ZB // L, fill_z, None)

    def fill_ones(i, _):
        ones1d[pl.ds(i * L, L)] = ones16
        return _

    lax.fori_loop(0, CH // L, fill_ones, None)

    def fill_ones_l(i, _):
        ones_l[pl.ds(i * L, L)] = ones16
        return _

    lax.fori_loop(0, CBL // L, fill_ones_l, None)

    zcopies = [
        pltpu.async_copy(zbuf, table.at[pl.ds(sid * Z_SPAN + j * ZB, ZB)], sem_z)
        for j in range(Z_SPAN // ZB)
    ]
    pending_in = pltpu.async_copy(inp.at[pl.ds(base, CHB)], inbufs[0], sem_in[0])
    for c in zcopies:
        c.wait()
    plsc.subcore_barrier()

    def tiled_bin(va, vb):
        return (
            ((va >> 3) << 13)
            | ((vb >> 7) << 10)
            | ((va & 7) << 7)
            | (vb & 127)
        )

    def compute_blocks(src, cdst):
        def blk_body(blk, _):
            for gg in range(8):
                va = src[blk, 0, pl.ds(gg * L, L)]
                vb = src[blk, 1, pl.ds(gg * L, L)]
                cdst[pl.ds(blk * 128 + gg * L, L)] = tiled_bin(va, vb)
            return _

        lax.fori_loop(0, CHB, blk_body, None)

    nreal = jnp.where(wid == NW - 1, TAILB, TAILB - REM)

    def compute_tail(src):
        def blk_body_l(blk, _):
            for gg in range(8):
                va = src[blk, 0, pl.ds(gg * L, L)]
                vb = src[blk, 1, pl.ds(gg * L, L)]
                c = jnp.where(blk < nreal, tiled_bin(va, vb), dummy16)
                cbuf_l[pl.ds(blk * 128 + gg * L, L)] = c
            return _

        lax.fori_loop(0, TAILB, blk_body_l, None)

    scat = [None, None]
    scat_l = None
    for ch in range(N_FULL + 1):
        cur = ch & 1
        nxt = 1 - cur
        next_in = None
        if ch + 1 < N_FULL:
            next_in = pltpu.async_copy(
                inp.at[pl.ds(base + (ch + 1) * CHB, CHB)], inbufs[nxt], sem_in[nxt]
            )
        elif ch + 1 == N_FULL:
            next_in = pltpu.async_copy(
                inp.at[pl.ds(base + N_FULL * CHB, TAILB)],
                inbufs[nxt].at[pl.ds(0, TAILB)],
                sem_in[nxt],
            )
        pending_in.wait()
        if ch < N_FULL:
            if scat[cur] is not None:
                scat[cur].wait()
            compute_blocks(inbufs[cur], cbufs[cur])
            scat[cur] = pltpu.async_copy(
                ones1d, table.at[cbufs[cur]], sem_s[cur], add=True
            )
        else:
            compute_tail(inbufs[cur])
            scat_l = pltpu.async_copy(ones_l, table.at[cbuf_l], sem_l, add=True)
        pending_in = next_in

    scat[0].wait()
    scat[1].wait()
    scat_l.wait()

    plsc.subcore_barrier()
    pltpu.sync_copy(
        table.at[pl.ds(sid * Z_SPAN, Z_SPAN)],
        out.at[pl.ds(cid * TBL + sid * Z_SPAN, Z_SPAN)],
    )


_sc_hist = functools.partial(
    pl.kernel,
    out_type=jax.ShapeDtypeStruct((NC * TBL,), jnp.float32),
    mesh=plsc.VectorSubcoreMesh(core_axis_name="c", subcore_axis_name="s"),
    compiler_params=pltpu.CompilerParams(needs_layout_passes=False),
    scratch_types=[
        pltpu.VMEM_SHARED((TBL,), jnp.float32),
        pltpu.VMEM((CHB, 2, 128), jnp.int32),
        pltpu.VMEM((CHB, 2, 128), jnp.int32),
        pltpu.VMEM((CH,), jnp.int32),
        pltpu.VMEM((CH,), jnp.int32),
        pltpu.VMEM((CH,), jnp.float32),
        pltpu.VMEM((CBL,), jnp.int32),
        pltpu.VMEM((CBL,), jnp.float32),
        pltpu.VMEM((ZB,), jnp.float32),
        pltpu.SemaphoreType.DMA,
        pltpu.SemaphoreType.DMA,
        pltpu.SemaphoreType.DMA,
        pltpu.SemaphoreType.DMA,
        pltpu.SemaphoreType.DMA,
        pltpu.SemaphoreType.DMA,
    ],
)(_hist_body)


def _finalize_body(p_ref, o_ref, a_ref):
    s = p_ref[0] + p_ref[1]
    v = s[:, :N]
    rs = jnp.sum(v, axis=1, keepdims=True)
    a_ref[...] = rs * (1.0 / NUM_SAMPLES)
    o_ref[...] = v / jnp.maximum(rs, 1.0)


_FROWS = 200


def _finalize(p3):
    return pl.pallas_call(
        _finalize_body,
        grid=(N // _FROWS,),
        in_specs=[pl.BlockSpec((NC, _FROWS, NBP), lambda i: (0, i, 0))],
        out_specs=[
            pl.BlockSpec((_FROWS, N), lambda i: (i, 0)),
            pl.BlockSpec((_FROWS, 1), lambda i: (i, 0)),
        ],
        out_shape=[
            jax.ShapeDtypeStruct((N, N), jnp.float32),
            jax.ShapeDtypeStruct((N, 1), jnp.float32),
        ],
    )(p3)


def kernel(inputs, vals):
    del vals
    view = inputs.reshape(NB, 128, 2).transpose(0, 2, 1)
    part = _sc_hist(view)
    p3 = (
        part.reshape(NC, N // 8, 8, NBP // 128, 128)
        .transpose(0, 1, 3, 2, 4)
        .reshape(NC, N, NBP)
    )
    pi_b_a, pi_a = _finalize(p3)
    return pi_a.reshape(N), pi_b_a
